# Initial kernel scaffold; baseline (speedup 1.0000x reference)
import functools

import jax
import jax.numpy as jnp
from jax import lax
from jax.experimental import pallas as pl
from jax.experimental.pallas import tpu as pltpu
from jax.experimental.pallas import tpu_sc as plsc

SCALE = jnp.float32(10.0)
NUM_ROWS = 16384
NUM_COLS = 26
EMBED_DIM = 32
B_TOTAL = NUM_ROWS * NUM_COLS
CHUNK = 128
NW = 32
PER_W = B_TOTAL // NW
NCH = PER_W // CHUNK

_mesh = plsc.VectorSubcoreMesh(core_axis_name="c", subcore_axis_name="s")


@functools.partial(
    pl.kernel,
    out_type=jax.ShapeDtypeStruct((B_TOTAL, EMBED_DIM), jnp.float32),
    mesh=_mesh,
    scratch_types=[
        pltpu.VMEM((NCH, CHUNK), jnp.int32),
        pltpu.VMEM((2, CHUNK, EMBED_DIM), jnp.float32),
        pltpu.SemaphoreType.DMA,
        pltpu.SemaphoreType.DMA,
    ],
)
def _gather_scale(x_hbm, table_hbm, out_hbm, idx_v, rows_v, gsem, ssem):
    wid = lax.axis_index("s") * 2 + lax.axis_index("c")
    row0 = wid * NCH
    base = wid * PER_W

    pltpu.sync_copy(x_hbm.at[pl.ds(row0, NCH)], idx_v)

    pltpu.async_copy(table_hbm.at[idx_v.at[0]], rows_v.at[0], gsem)

    def step(c, carry):
        b = lax.rem(c, 2)
        nb = 1 - b

        @pl.when(c >= 1)
        def _wait_store():
            pltpu.make_async_copy(
                rows_v.at[nb], out_hbm.at[pl.ds(base, CHUNK)], ssem
            ).wait()

        @pl.when(c + 1 < NCH)
        def _next_gather():
            pltpu.async_copy(table_hbm.at[idx_v.at[c + 1]], rows_v.at[nb], gsem)

        pltpu.make_async_copy(
            table_hbm.at[idx_v.at[c]], rows_v.at[b], gsem
        ).wait()

        def srow(r, rcarry):
            for k in range(EMBED_DIM // 16):
                sl = pl.ds(k * 16, 16)
                rows_v[b, r, sl] = rows_v[b, r, sl] * SCALE
            return rcarry

        lax.fori_loop(0, CHUNK, srow, 0)

        pltpu.async_copy(
            rows_v.at[b], out_hbm.at[pl.ds(base + c * CHUNK, CHUNK)], ssem
        )
        return carry

    lax.fori_loop(0, NCH, step, 0)

    pltpu.make_async_copy(
        rows_v.at[0], out_hbm.at[pl.ds(base, CHUNK)], ssem
    ).wait()


@jax.jit
def kernel(x, table):
    x2d = x.reshape(B_TOTAL // CHUNK, CHUNK)
    out = _gather_scale(x2d, table)
    return out.reshape(NUM_ROWS, NUM_COLS, EMBED_DIM)


# speedup vs baseline: 1.1964x; 1.1964x over previous
import functools

import jax
import jax.numpy as jnp
from jax import lax
from jax.experimental import pallas as pl
from jax.experimental.pallas import tpu as pltpu
from jax.experimental.pallas import tpu_sc as plsc

SCALE = 10.0
NUM_ROWS = 16384
NUM_COLS = 26
EMBED_DIM = 32
B_TOTAL = NUM_ROWS * NUM_COLS
CHUNK = 128
NW = 32
PER_W = B_TOTAL // NW
NCH = PER_W // CHUNK

_mesh = plsc.VectorSubcoreMesh(core_axis_name="c", subcore_axis_name="s")


@functools.partial(
    pl.kernel,
    out_type=jax.ShapeDtypeStruct((B_TOTAL, EMBED_DIM), jnp.float32),
    mesh=_mesh,
    scratch_types=[
        pltpu.VMEM((NCH, CHUNK), jnp.int32),
        pltpu.VMEM((2, CHUNK, EMBED_DIM), jnp.float32),
        pltpu.SemaphoreType.DMA,
        pltpu.SemaphoreType.DMA,
    ],
    compiler_params=pltpu.CompilerParams(use_tc_tiling_on_sc=False),
)
def _gather_scale(x_hbm, table_hbm, out_hbm, idx_v, rows_v, gsem, ssem):
    wid = lax.axis_index("s") * 2 + lax.axis_index("c")
    row0 = wid * NCH
    base = wid * PER_W

    pltpu.sync_copy(x_hbm.at[pl.ds(row0, NCH)], idx_v)

    pltpu.async_copy(table_hbm.at[idx_v.at[0]], rows_v.at[0], gsem)

    def step(c, carry):
        b = lax.rem(c, 2)
        nb = 1 - b

        @pl.when(c >= 1)
        def _wait_store():
            pltpu.make_async_copy(
                rows_v.at[nb], out_hbm.at[pl.ds(base, CHUNK)], ssem
            ).wait()

        @pl.when(c + 1 < NCH)
        def _next_gather():
            pltpu.async_copy(table_hbm.at[idx_v.at[c + 1]], rows_v.at[nb], gsem)

        pltpu.make_async_copy(
            table_hbm.at[idx_v.at[c]], rows_v.at[b], gsem
        ).wait()

        def srow(r, rcarry):
            for k in range(EMBED_DIM // 16):
                sl = pl.ds(k * 16, 16)
                rows_v[b, r, sl] = rows_v[b, r, sl] * SCALE
            return rcarry

        lax.fori_loop(0, CHUNK, srow, 0)

        pltpu.async_copy(
            rows_v.at[b], out_hbm.at[pl.ds(base + c * CHUNK, CHUNK)], ssem
        )
        return carry

    lax.fori_loop(0, NCH, step, 0)

    pltpu.make_async_copy(
        rows_v.at[0], out_hbm.at[pl.ds(base, CHUNK)], ssem
    ).wait()


@jax.jit
def kernel(x, table):
    x2d = x.reshape(B_TOTAL // CHUNK, CHUNK)
    out = _gather_scale(x2d, table)
    return out.reshape(NUM_ROWS, NUM_COLS, EMBED_DIM)


# speedup vs baseline: 1.3460x; 1.1250x over previous
import functools

import jax
import jax.numpy as jnp
from jax import lax
from jax.experimental import pallas as pl
from jax.experimental.pallas import tpu as pltpu
from jax.experimental.pallas import tpu_sc as plsc

SCALE = 10.0
NUM_ROWS = 16384
NUM_COLS = 26
EMBED_DIM = 32
B_TOTAL = NUM_ROWS * NUM_COLS
CHUNK = 128
NW = 32
PER_W = B_TOTAL // NW
NCH = PER_W // CHUNK

_mesh = plsc.VectorSubcoreMesh(core_axis_name="c", subcore_axis_name="s")


@functools.partial(
    pl.kernel,
    out_type=jax.ShapeDtypeStruct((B_TOTAL, EMBED_DIM), jnp.float32),
    mesh=_mesh,
    scratch_types=[
        pltpu.VMEM((NCH, CHUNK), jnp.int32),
        pltpu.VMEM((2, CHUNK, EMBED_DIM), jnp.float32),
        pltpu.SemaphoreType.DMA,
        pltpu.SemaphoreType.DMA,
    ],
    compiler_params=pltpu.CompilerParams(use_tc_tiling_on_sc=False),
)
def _gather_scale(x_hbm, table_hbm, out_hbm, idx_v, rows_v, gsem, ssem):
    wid = lax.axis_index("s") * 2 + lax.axis_index("c")
    row0 = wid * NCH
    base = wid * PER_W

    pltpu.sync_copy(x_hbm.at[pl.ds(row0, NCH)], idx_v)

    pltpu.async_copy(table_hbm.at[idx_v.at[0]], rows_v.at[0], gsem)

    def step(c, carry):
        b = lax.rem(c, 2)
        nb = 1 - b

        @pl.when(c >= 1)
        def _wait_store():
            pltpu.make_async_copy(
                rows_v.at[nb], out_hbm.at[pl.ds(base, CHUNK)], ssem
            ).wait()

        @pl.when(c + 1 < NCH)
        def _next_gather():
            pltpu.async_copy(table_hbm.at[idx_v.at[c + 1]], rows_v.at[nb], gsem)

        pltpu.make_async_copy(
            table_hbm.at[idx_v.at[c]], rows_v.at[b], gsem
        ).wait()

        @plsc.parallel_loop(0, CHUNK, unroll=8)
        def _scale(r):
            for k in range(EMBED_DIM // 16):
                sl = pl.ds(k * 16, 16)
                rows_v[b, r, sl] = rows_v[b, r, sl] * SCALE

        pltpu.async_copy(
            rows_v.at[b], out_hbm.at[pl.ds(base + c * CHUNK, CHUNK)], ssem
        )
        return carry

    lax.fori_loop(0, NCH, step, 0)

    pltpu.make_async_copy(
        rows_v.at[0], out_hbm.at[pl.ds(base, CHUNK)], ssem
    ).wait()


@jax.jit
def kernel(x, table):
    x2d = x.reshape(B_TOTAL // CHUNK, CHUNK)
    out = _gather_scale(x2d, table)
    return out.reshape(NUM_ROWS, NUM_COLS, EMBED_DIM)


# speedup vs baseline: 1.4004x; 1.0405x over previous
import functools

import jax
import jax.numpy as jnp
from jax import lax
from jax.experimental import pallas as pl
from jax.experimental.pallas import tpu as pltpu
from jax.experimental.pallas import tpu_sc as plsc

SCALE = 10.0
NUM_ROWS = 16384
NUM_COLS = 26
EMBED_DIM = 32
B_TOTAL = NUM_ROWS * NUM_COLS
CHUNK = 128
NW = 32
PER_W = B_TOTAL // NW
NCH = PER_W // CHUNK
NBUF = 8

_mesh = plsc.VectorSubcoreMesh(core_axis_name="c", subcore_axis_name="s")


@functools.partial(
    pl.kernel,
    out_type=jax.ShapeDtypeStruct((B_TOTAL, EMBED_DIM), jnp.float32),
    mesh=_mesh,
    scratch_types=[
        pltpu.VMEM((NCH, CHUNK), jnp.int32),
        pltpu.VMEM((NBUF, CHUNK, EMBED_DIM), jnp.float32),
        pltpu.SemaphoreType.DMA,
        pltpu.SemaphoreType.DMA,
    ],
    compiler_params=pltpu.CompilerParams(use_tc_tiling_on_sc=False),
)
def _gather_scale(x_hbm, table_hbm, out_hbm, idx_v, rows_v, gsem, ssem):
    wid = lax.axis_index("s") * 2 + lax.axis_index("c")
    row0 = wid * NCH
    base = wid * PER_W

    pltpu.sync_copy(x_hbm.at[pl.ds(row0, NCH)], idx_v)

    for p in range(NBUF - 1):
        pltpu.async_copy(table_hbm.at[idx_v.at[p]], rows_v.at[p], gsem)

    def step(c, carry):
        b = lax.rem(c, NBUF)
        pb = lax.rem(c + NBUF - 1, NBUF)

        @pl.when(c >= 1)
        def _wait_store():
            pltpu.make_async_copy(
                rows_v.at[pb], out_hbm.at[pl.ds(base, CHUNK)], ssem
            ).wait()

        @pl.when(c + NBUF - 1 < NCH)
        def _next_gather():
            pltpu.async_copy(
                table_hbm.at[idx_v.at[c + NBUF - 1]], rows_v.at[pb], gsem
            )

        pltpu.make_async_copy(
            table_hbm.at[idx_v.at[c]], rows_v.at[b], gsem
        ).wait()

        @plsc.parallel_loop(0, CHUNK, unroll=8)
        def _scale(r):
            for k in range(EMBED_DIM // 16):
                sl = pl.ds(k * 16, 16)
                rows_v[b, r, sl] = rows_v[b, r, sl] * SCALE

        pltpu.async_copy(
            rows_v.at[b], out_hbm.at[pl.ds(base + c * CHUNK, CHUNK)], ssem
        )
        return carry

    lax.fori_loop(0, NCH, step, 0)

    pltpu.make_async_copy(
        rows_v.at[0], out_hbm.at[pl.ds(base, CHUNK)], ssem
    ).wait()


@jax.jit
def kernel(x, table):
    x2d = x.reshape(B_TOTAL // CHUNK, CHUNK)
    out = _gather_scale(x2d, table)
    return out.reshape(NUM_ROWS, NUM_COLS, EMBED_DIM)


# speedup vs baseline: 1.4057x; 1.0038x over previous
import functools

import jax
import jax.numpy as jnp
from jax import lax
from jax.experimental import pallas as pl
from jax.experimental.pallas import tpu as pltpu
from jax.experimental.pallas import tpu_sc as plsc

SCALE = 10.0
NUM_ROWS = 16384
NUM_COLS = 26
EMBED_DIM = 32
NW = 32
ROWS_W = NUM_ROWS // NW
PER_W = ROWS_W * NUM_COLS
RB = 4
CHUNK = RB * NUM_COLS
NCH = ROWS_W // RB
NBUF = 8
NSBUF = 4

_mesh = plsc.VectorSubcoreMesh(core_axis_name="c", subcore_axis_name="s")


@functools.partial(
    pl.kernel,
    out_type=jax.ShapeDtypeStruct((NUM_ROWS, NUM_COLS, EMBED_DIM), jnp.float32),
    mesh=_mesh,
    scratch_types=[
        pltpu.VMEM((PER_W,), jnp.int32),
        pltpu.VMEM((NBUF, CHUNK, EMBED_DIM), jnp.float32),
        pltpu.VMEM((NSBUF, RB, NUM_COLS, EMBED_DIM), jnp.float32),
        pltpu.SemaphoreType.DMA,
        pltpu.SemaphoreType.DMA,
    ],
    compiler_params=pltpu.CompilerParams(use_tc_tiling_on_sc=False),
)
def _gather_scale(x_hbm, table_hbm, out_hbm, idx_v, gbuf, sbuf, gsem, ssem):
    wid = lax.axis_index("s") * 2 + lax.axis_index("c")
    row0 = wid * ROWS_W

    pltpu.sync_copy(x_hbm.at[pl.ds(wid * PER_W, PER_W)], idx_v)

    for p in range(NBUF - 1):
        pltpu.async_copy(
            table_hbm.at[idx_v.at[pl.ds(p * CHUNK, CHUNK)]], gbuf.at[p], gsem
        )

    def step(c, carry):
        b = lax.rem(c, NBUF)
        pb = lax.rem(c + NBUF - 1, NBUF)
        sb = lax.rem(c, NSBUF)

        @pl.when(c + NBUF - 1 < NCH)
        def _next_gather():
            pltpu.async_copy(
                table_hbm.at[idx_v.at[pl.ds((c + NBUF - 1) * CHUNK, CHUNK)]],
                gbuf.at[pb],
                gsem,
            )

        @pl.when(c >= NSBUF)
        def _wait_store():
            pltpu.make_async_copy(
                sbuf.at[sb], out_hbm.at[pl.ds(row0, RB)], ssem
            ).wait()

        pltpu.make_async_copy(
            table_hbm.at[idx_v.at[pl.ds(c * CHUNK, CHUNK)]], gbuf.at[b], gsem
        ).wait()

        @plsc.parallel_loop(0, NUM_COLS, unroll=13)
        def _scale(col):
            for r in range(RB):
                for k in range(EMBED_DIM // 16):
                    sl = pl.ds(k * 16, 16)
                    sbuf[sb, r, col, sl] = gbuf[b, r * NUM_COLS + col, sl] * SCALE

        pltpu.async_copy(
            sbuf.at[sb], out_hbm.at[pl.ds(row0 + c * RB, RB)], ssem
        )
        return carry

    lax.fori_loop(0, NCH, step, 0)

    for _ in range(NSBUF):
        pltpu.make_async_copy(
            sbuf.at[0], out_hbm.at[pl.ds(row0, RB)], ssem
        ).wait()


@jax.jit
def kernel(x, table):
    return _gather_scale(x.reshape(-1), table)
